# baseline (device time: 36579 ns/iter reference)
import jax
import jax.numpy as jnp
from jax import lax
from jax.experimental import pallas as pl
from jax.experimental.pallas import tpu as pltpu

N_DEV = 32
N_HALF = 2


def kernel(x):
    m, n = x.shape
    c = m // N_DEV
    hc = c // N_HALF

    def body(x_ref, out_ref, rs_recv, rs_send_sems, rs_recv_sems,
             ag_send_sems, ag_recv_sems):
        my = lax.axis_index("i")

        barrier_sem = pltpu.get_barrier_semaphore()
        for k in range(1, N_DEV):
            pl.semaphore_signal(
                barrier_sem, inc=1,
                device_id=((my + k) % N_DEV,),
                device_id_type=pl.DeviceIdType.MESH,
            )
        pl.semaphore_wait(barrier_sem, N_DEV - 1)

        def rs_desc(p, h):
            return pltpu.make_async_remote_copy(
                src_ref=x_ref.at[pl.ds(p * c + h * hc, hc), :],
                dst_ref=rs_recv.at[my, pl.ds(h * hc, hc), :],
                send_sem=rs_send_sems.at[p, h],
                recv_sem=rs_recv_sems.at[my, h],
                device_id=(p,),
                device_id_type=pl.DeviceIdType.MESH,
            )

        def rs_wait_desc(p, h):
            return pltpu.make_async_remote_copy(
                src_ref=x_ref.at[pl.ds(p * c + h * hc, hc), :],
                dst_ref=rs_recv.at[p, pl.ds(h * hc, hc), :],
                send_sem=rs_send_sems.at[p, h],
                recv_sem=rs_recv_sems.at[p, h],
                device_id=(p,),
                device_id_type=pl.DeviceIdType.MESH,
            )

        def ag_desc(p, h):
            return pltpu.make_async_remote_copy(
                src_ref=out_ref.at[pl.ds(my * c + h * hc, hc), :],
                dst_ref=out_ref.at[pl.ds(my * c + h * hc, hc), :],
                send_sem=ag_send_sems.at[p, h],
                recv_sem=ag_recv_sems.at[my, h],
                device_id=(p,),
                device_id_type=pl.DeviceIdType.MESH,
            )

        def ag_wait_desc(p, h):
            return pltpu.make_async_remote_copy(
                src_ref=out_ref.at[pl.ds(my * c + h * hc, hc), :],
                dst_ref=out_ref.at[pl.ds(p * c + h * hc, hc), :],
                send_sem=ag_send_sems.at[p, h],
                recv_sem=ag_recv_sems.at[p, h],
                device_id=(p,),
                device_id_type=pl.DeviceIdType.MESH,
            )

        for h in range(N_HALF):
            for k in range(1, N_DEV):
                rs_desc((my + k) % N_DEV, h).start()
        rs_recv[pl.ds(my, 1), :, :] = x_ref[pl.ds(my * c, c), :][None]

        for h in range(N_HALF):
            for k in range(1, N_DEV):
                rs_wait_desc((my + k) % N_DEV, h).wait_recv()
            reduced = jnp.sum(rs_recv[:, h * hc:(h + 1) * hc, :], axis=0)
            out_ref[pl.ds(my * c + h * hc, hc), :] = reduced
            for k in range(1, N_DEV):
                ag_desc((my + k) % N_DEV, h).start()

        for h in range(N_HALF):
            for k in range(1, N_DEV):
                ag_wait_desc((my + k) % N_DEV, h).wait_recv()
        for h in range(N_HALF):
            for k in range(1, N_DEV):
                p = (my + k) % N_DEV
                rs_wait_desc(p, h).wait_send()
                ag_wait_desc(p, h).wait_send()

    return pl.pallas_call(
        body,
        out_shape=jax.ShapeDtypeStruct((m, n), x.dtype),
        in_specs=[pl.BlockSpec(memory_space=pltpu.VMEM)],
        out_specs=pl.BlockSpec(memory_space=pltpu.VMEM),
        scratch_shapes=[
            pltpu.VMEM((N_DEV, c, n), x.dtype),
            pltpu.SemaphoreType.DMA((N_DEV, N_HALF)),
            pltpu.SemaphoreType.DMA((N_DEV, N_HALF)),
            pltpu.SemaphoreType.DMA((N_DEV, N_HALF)),
            pltpu.SemaphoreType.DMA((N_DEV, N_HALF)),
        ],
        compiler_params=pltpu.CompilerParams(collective_id=0),
    )(x)
